# baseline (speedup 1.0000x reference)
import dataclasses

import jax
import jax.numpy as jnp
from jax import lax
from jax.experimental import pallas as pl
from jax.experimental.pallas import tpu as pltpu
from jax.experimental.pallas import tpu_sc as plsc

NUM_IDX = 16384
EMB = 64
FUSED = 128
NUM_CORES = 2
NUM_SUBCORES = 16
NUM_WORKERS = NUM_CORES * NUM_SUBCORES
B_PER_W = NUM_IDX // NUM_WORKERS
IDX_CHUNK = 128
NUM_CHUNKS = B_PER_W // IDX_CHUNK
HALF = B_PER_W // 2
LANES = 16


def kernel(publisher_idx, embedding_publisher):
    idx = publisher_idx.astype(jnp.int32)
    fused_idx = (idx >> 1).reshape(NUM_WORKERS, NUM_CHUNKS, IDX_CHUNK)
    offs = ((idx & 1) * EMB).reshape(NUM_WORKERS, B_PER_W)
    table_f = embedding_publisher.reshape(-1, FUSED)
    mesh = plsc.VectorSubcoreMesh(core_axis_name="c", subcore_axis_name="s")
    cp = pltpu.CompilerParams()
    if "needs_layout_passes" in pltpu.CompilerParams.__dataclass_fields__:
        cp = dataclasses.replace(cp, needs_layout_passes=False)

    @pl.kernel(
        compiler_params=cp,
        out_type=jax.ShapeDtypeStruct((NUM_IDX, EMB), embedding_publisher.dtype),
        mesh=mesh,
        scratch_types=[
            pltpu.VMEM((NUM_CHUNKS, IDX_CHUNK), jnp.int32),
            pltpu.VMEM((B_PER_W,), jnp.int32),
            pltpu.VMEM((HALF, FUSED), jnp.float32),
            pltpu.VMEM((HALF, FUSED), jnp.float32),
            pltpu.VMEM((HALF, EMB), jnp.float32),
            pltpu.SemaphoreType.DMA,
            pltpu.SemaphoreType.DMA,
        ],
    )
    def gather_kernel(
        table_hbm,
        fidx_hbm,
        offs_hbm,
        out_hbm,
        fidx_v,
        offs_v,
        rows_a,
        rows_b,
        out_v,
        sem_a,
        sem_b,
    ):
        wid = lax.axis_index("s") * NUM_CORES + lax.axis_index("c")
        base = wid * B_PER_W
        pltpu.sync_copy(fidx_hbm.at[wid], fidx_v)
        pltpu.sync_copy(offs_hbm.at[wid], offs_v)

        for j, (buf, sem) in enumerate(
            [(rows_a, sem_a), (rows_a, sem_a), (rows_b, sem_b), (rows_b, sem_b)]
        ):
            pltpu.async_copy(
                table_hbm.at[fidx_v.at[j]],
                buf.at[pl.ds((j % 2) * IDX_CHUNK, IDX_CHUNK)],
                sem,
            )

        for h, (buf, sem) in enumerate([(rows_a, sem_a), (rows_b, sem_b)]):
            pltpu.make_async_copy(table_hbm.at[pl.ds(0, HALF)], buf, sem).wait()

            @pl.loop(0, HALF // LANES)
            def _(g, buf=buf, h=h):
                off_vec = offs_v[pl.ds(h * HALF + g * LANES, LANES)]
                for k in range(LANES):
                    row = g * LANES + k
                    off = off_vec[k]
                    for t in range(EMB // LANES):
                        out_v[row, pl.ds(t * LANES, LANES)] = buf[
                            row, pl.ds(off + t * LANES, LANES)
                        ]

            pltpu.sync_copy(out_v, out_hbm.at[pl.ds(base + h * HALF, HALF)])

    return gather_kernel(table_f, fused_idx, offs)
